# Initial kernel scaffold; baseline (speedup 1.0000x reference)
import jax
import jax.numpy as jnp
from jax.experimental import pallas as pl


def kernel(batch_features, emb_table):
    raise NotImplementedError("write your pallas kernel here")


# speedup vs baseline: 3.2718x; 3.2718x over previous
import functools

import jax
import jax.numpy as jnp
from jax import lax
from jax.experimental import pallas as pl
from jax.experimental.pallas import tpu as pltpu, tpu_sc as plsc

B, S = 1024, 50
MAXW = 20
EMB = 32
FEAT = 64
ROW = 100
WPT = 22
N_TOK = B * S

NC, NS = 2, 16
NW = NC * NS
TPW = N_TOK // NW
T = 64
NCHUNK = TPW // T
G = 128
NG = T * WPT // G

_mesh = plsc.VectorSubcoreMesh(core_axis_name="c", subcore_axis_name="s")


@functools.partial(
    pl.kernel,
    out_type=jax.ShapeDtypeStruct((N_TOK * WPT, EMB), jnp.float32),
    mesh=_mesh,
    scratch_types=[
        pltpu.VMEM((T, ROW), jnp.float32),
        pltpu.VMEM((T * WPT + 16,), jnp.int32),
        pltpu.VMEM((T * WPT, EMB), jnp.float32),
        pltpu.SemaphoreType.DMA,
    ],
    compiler_params=pltpu.CompilerParams(use_tc_tiling_on_sc=False),
)
def _encode_sc(bf_hbm, table_hbm, out_hbm, bf_v, idx_v, emb_v, sem):
    wid = lax.axis_index("s") * NC + lax.axis_index("c")
    lane = lax.broadcasted_iota(jnp.int32, (16,), 0)

    def chunk_body(c, carry):
        tok0 = wid * TPW + c * T
        pltpu.sync_copy(bf_hbm.at[pl.ds(tok0, T)], bf_v)

        def idx_body(t, _):
            v0 = bf_v[t, pl.ds(0, 16)].astype(jnp.int32)
            v1 = bf_v[t, pl.ds(16, 16)].astype(jnp.int32)
            v1 = jnp.where(lane < MAXW - 16, v1, 0)
            idx_v[pl.ds(WPT * t, 16)] = v0
            idx_v[pl.ds(WPT * t + 16, 16)] = v1
            return _

        lax.fori_loop(0, T, idx_body, None)

        copies = [
            pltpu.async_copy(
                table_hbm.at[idx_v.at[pl.ds(g * G, G)]],
                emb_v.at[pl.ds(g * G, G)],
                sem,
            )
            for g in range(NG)
        ]
        for cp in copies:
            cp.wait()

        def fix_body(t, _):
            for i in range(2):
                for m in range(2):
                    emb_v[WPT * t + MAXW + i, pl.ds(m * 16, 16)] = bf_v[
                        t, pl.ds(MAXW + (2 * i + m) * 16, 16)
                    ]
            return _

        lax.fori_loop(0, T, fix_body, None)

        pltpu.sync_copy(emb_v, out_hbm.at[pl.ds(tok0 * WPT, T * WPT)])
        return carry

    lax.fori_loop(0, NCHUNK, chunk_body, None)


def kernel(batch_features, emb_table):
    bf = batch_features.reshape(N_TOK, ROW)
    out = _encode_sc(bf, emb_table)
    return out.reshape(B, S, WPT * EMB)
